# Initial kernel scaffold; baseline (speedup 1.0000x reference)
import jax
import jax.numpy as jnp
from jax.experimental import pallas as pl


def kernel(box_cls, gau_logits):
    raise NotImplementedError("write your pallas kernel here")


# speedup vs baseline: 11.5287x; 11.5287x over previous
import functools

import jax
import jax.numpy as jnp
from jax import lax
from jax.experimental import pallas as pl
from jax.experimental.pallas import tpu as pltpu
from jax.experimental.pallas import tpu_sc as plsc

N = 2
C = 80
H = 160
W = 160
HW = H * W
TOT = C * HW
PRE = 0.05
K = 1000
STEP = 8
SIGMA2 = 0.125

NB = 4096
NTILES = 16
PER_TILE = TOT // NTILES
CH = 16000
NCH = PER_TILE // CH
TCAP = 128
GCAP = NTILES * TCAP
DUMP = K


def _dense_body(box_ref, gau_ref, out_ref):
    g = gau_ref[...]
    bx = box_ref[...]
    gp = 1.0 / (1.0 + jnp.exp(-g))
    bp = 1.0 / (1.0 + jnp.exp(-bx))
    m1 = jnp.maximum(gp, jnp.maximum(pltpu.roll(gp, 1, 1), pltpu.roll(gp, H - 1, 1)))
    mp = jnp.maximum(m1, jnp.maximum(pltpu.roll(m1, 1, 2), pltpu.roll(m1, W - 1, 2)))
    iy = lax.broadcasted_iota(jnp.int32, g.shape, 1)
    ix = lax.broadcasted_iota(jnp.int32, g.shape, 2)
    interior = (iy > 0) & (iy < H - 1) & (ix > 0) & (ix < W - 1)
    keep = (gp == mp) & (bp > PRE) & interior
    out_ref[...] = jnp.where(keep, jnp.sqrt(gp * bp), jnp.float32(-1.0))


def _masked_scores(box_cls, gau_logits):
    x3 = box_cls.reshape(N * C, H, W)
    g3 = gau_logits.reshape(N * C, H, W)
    m = pl.pallas_call(
        _dense_body,
        grid=(20,),
        in_specs=[pl.BlockSpec((8, H, W), lambda i: (i, 0, 0))] * 2,
        out_specs=pl.BlockSpec((8, H, W), lambda i: (i, 0, 0)),
        out_shape=jax.ShapeDtypeStruct((N * C, H, W), jnp.float32),
    )(x3, g3)
    return m.reshape(N, TOT)


def _sc_body(m_hbm, gau_hbm, boxes_hbm, scores_hbm, labels_hbm,
             chunk_v, hist_v, red_v, redsum_v, hist_full_v, edge_v,
             cs_v, ci_v, all_s, all_i, ridx_v, gidx_v, gbuf_v,
             sc_v, lb_v, b0_v, b1_v, b2_v, b3_v, bidx_v, zbi_v, zbx_v,
             hists_s, hist_red_s, cands_s, candi_s, edge_s,
             stage_sc, stage_lb, stage_bx, sem):
    b = lax.axis_index("c")
    sid = lax.axis_index("s")
    base = sid * PER_TILE
    z16i = jnp.zeros((16,), jnp.int32)
    z16f = jnp.zeros((16,), jnp.float32)
    ones16 = jnp.ones((16,), jnp.int32)
    iota16 = lax.broadcasted_iota(jnp.int32, (16,), 0)
    fNB = jnp.float32(NB)

    @pl.when(sid == 0)
    def _p0():
        def zi(i, _):
            zbi_v[pl.ds(i * 16, 16)] = z16i
            return 0
        lax.fori_loop(0, 1008 // 16, zi, 0)

        def zf(i, _):
            zbx_v[pl.ds(i * 16, 16)] = z16f
            return 0
        lax.fori_loop(0, 4032 // 16, zf, 0)
        pltpu.sync_copy(zbx_v.at[pl.ds(0, 1008)], stage_sc)
        pltpu.sync_copy(zbi_v, stage_lb)
        pltpu.sync_copy(zbx_v, stage_bx)

    def zh(i, _):
        hist_v[pl.ds(i * 16, 16)] = z16i
        return 0
    lax.fori_loop(0, NB // 16, zh, 0)

    def hist_chunk(k, _):
        pltpu.sync_copy(m_hbm.at[b, pl.ds(base + k * CH, CH)], chunk_v)

        def jloop(j, _2):
            for u in range(8):
                s = chunk_v[pl.ds((j * 8 + u) * 16, 16)]
                bins = jnp.clip((s * fNB).astype(jnp.int32), 0, NB - 1)
                plsc.addupdate_scatter(hist_v, [bins], ones16, mask=s >= 0.0)
            return 0
        lax.fori_loop(0, CH // 128, jloop, 0)
        return 0
    lax.fori_loop(0, NCH, hist_chunk, 0)
    pltpu.sync_copy(hist_v, hists_s.at[sid])
    plsc.subcore_barrier()

    for t in range(NTILES):
        pltpu.sync_copy(hists_s.at[t, pl.ds(sid * 256, 256)], red_v.at[t])

    def red_loop(j, _):
        acc = z16i
        for t in range(NTILES):
            acc = acc + red_v[t, pl.ds(j * 16, 16)]
        redsum_v[pl.ds(j * 16, 16)] = acc
        return 0
    lax.fori_loop(0, 256 // 16, red_loop, 0)
    pltpu.sync_copy(redsum_v, hist_red_s.at[pl.ds(sid * 256, 256)])
    plsc.subcore_barrier()

    @pl.when(sid == 0)
    def _p2b():
        pltpu.sync_copy(hist_red_s, hist_full_v)

        def scan_body(j, carry):
            run, found, bst = carry
            vidx = (NB // 16 - 1) - j
            v = hist_full_v[pl.ds(vidx * 16, 16)]
            vd = lax.rev(v, (0,))
            cum = plsc.cumsum(vd) + run
            msk = cum >= K
            anym = jnp.any(msk)
            ffs = jnp.max(plsc.all_reduce_ffs(msk))
            cand = vidx * 16 + 15 - ffs
            take = jnp.logical_and(jnp.logical_not(found), anym)
            return (run + jnp.sum(v),
                    jnp.logical_or(found, anym),
                    jnp.where(take, cand, bst))
        _, found, bst = lax.fori_loop(
            0, NB // 16, scan_body,
            (jnp.int32(0), jnp.bool_(False), jnp.int32(0)))
        bst = jnp.where(found, bst, jnp.int32(0))
        edge_v[...] = jnp.full((16,), 0, jnp.int32) + bst
        pltpu.sync_copy(edge_v, edge_s)
    plsc.subcore_barrier()
    pltpu.sync_copy(edge_s, edge_v)
    bstar = jnp.max(edge_v[...])

    for t in range((TCAP + 16) // 16):
        cs_v[pl.ds(t * 16, 16)] = z16f - 1.0
        ci_v[pl.ds(t * 16, 16)] = z16i

    def comp_chunk(k, wcount):
        pltpu.sync_copy(m_hbm.at[b, pl.ds(base + k * CH, CH)], chunk_v)

        def jloop(j, wc):
            for u in range(4):
                off = (j * 4 + u) * 16
                s = chunk_v[pl.ds(off, 16)]
                bins = jnp.clip((s * fNB).astype(jnp.int32), 0, NB - 1)
                msk = jnp.logical_and(s >= 0.0, bins >= bstar)
                idxv = base + k * CH + off + iota16
                wclip = jnp.minimum(wc, TCAP)
                plsc.store_compressed(cs_v.at[pl.ds(wclip, 16)], s, mask=msk)
                plsc.store_compressed(ci_v.at[pl.ds(wclip, 16)], idxv, mask=msk)
                wc = wc + jnp.max(plsc.all_reduce_population_count(msk))
            return wc
        return lax.fori_loop(0, CH // 64, jloop, wcount)
    lax.fori_loop(0, NCH, comp_chunk, jnp.int32(0))
    pltpu.sync_copy(cs_v.at[pl.ds(0, TCAP)], cands_s.at[pl.ds(sid * TCAP, TCAP)])
    pltpu.sync_copy(ci_v.at[pl.ds(0, TCAP)], candi_s.at[pl.ds(sid * TCAP, TCAP)])
    plsc.subcore_barrier()

    pltpu.sync_copy(cands_s, all_s)
    pltpu.sync_copy(candi_s, all_i)

    def my_loop(tmy, _):
        sjv = cs_v[pl.ds(tmy * 16, 16)]
        ijv = ci_v[pl.ds(tmy * 16, 16)]

        def other_loop(t, rankv):
            va = all_s[pl.ds(t * 16, 16)]
            vi = all_i[pl.ds(t * 16, 16)]
            for lane in range(16):
                s_i = va[lane]
                i_i = vi[lane]
                better = jnp.logical_or(
                    s_i > sjv, jnp.logical_and(s_i == sjv, i_i < ijv))
                rankv = rankv + jnp.where(better, 1, 0)
            return rankv
        rankv = lax.fori_loop(0, GCAP // 16, other_loop, z16i)
        rankv = jnp.where(
            jnp.logical_and(sjv >= 0.0, rankv < K), rankv, jnp.int32(DUMP))
        ridx_v[pl.ds(tmy * 16, 16)] = rankv
        return 0
    lax.fori_loop(0, TCAP // 16, my_loop, 0)

    for t in range(TCAP // 16):
        o = t * 16
        sv = cs_v[pl.ds(o, 16)]
        iv = ci_v[pl.ds(o, 16)]
        gidx_v[pl.ds(o, 16)] = jnp.where(sv >= 0.0, iv, 0) + b * TOT
        sc_v[pl.ds(o, 16)] = sv
    pltpu.async_copy(gau_hbm.at[gidx_v], gbuf_v, sem).wait()

    for t in range(TCAP // 16):
        o = t * 16
        g = gbuf_v[pl.ds(o, 16)]
        iv = ci_v[pl.ds(o, 16)]
        u = jnp.exp(-jnp.abs(g))
        z = u / (2.0 + u)
        z2 = z * z
        l1p = 2.0 * z * (1.0 + z2 * (jnp.float32(1.0 / 3.0) + z2 * (
            jnp.float32(0.2) + z2 * jnp.float32(1.0 / 7.0))))
        li = (jnp.maximum(-g, 0.0) + l1p) * jnp.float32(SIGMA2)
        xc = jnp.maximum(li, jnp.float32(1e-12))
        bi = plsc.bitcast(xc, jnp.int32)
        yi = jnp.int32(0x5F3759DF) - lax.shift_right_arithmetic(bi, 1)
        y = plsc.bitcast(yi, jnp.float32)
        for _ in range(3):
            y = y * (1.5 - 0.5 * xc * y * y)
        halfw = xc * y * jnp.float32(STEP) + 0.5
        ch = iv // HW
        rem = iv % HW
        yy = rem // W
        xx = rem % W
        cx = xx.astype(jnp.float32) * STEP + jnp.float32((STEP - 1) / 2.0)
        cy = yy.astype(jnp.float32) * STEP + jnp.float32((STEP - 1) / 2.0)
        b0_v[pl.ds(o, 16)] = cx - halfw
        b1_v[pl.ds(o, 16)] = cy - halfw
        b2_v[pl.ds(o, 16)] = cx + halfw
        b3_v[pl.ds(o, 16)] = cy + halfw
        lb_v[pl.ds(o, 16)] = ch + 1

    pltpu.sync_copy(sc_v, stage_sc.at[ridx_v])
    pltpu.sync_copy(lb_v, stage_lb.at[ridx_v])
    for kc, bv in ((0, b0_v), (1, b1_v), (2, b2_v), (3, b3_v)):
        for t in range(TCAP // 16):
            o = t * 16
            bidx_v[pl.ds(o, 16)] = ridx_v[pl.ds(o, 16)] * 4 + kc
        pltpu.sync_copy(bv, stage_bx.at[bidx_v])
    plsc.subcore_barrier()

    @pl.when(sid == 0)
    def _p5():
        pltpu.sync_copy(stage_sc.at[pl.ds(0, K)], scores_hbm.at[b])
        pltpu.sync_copy(stage_lb.at[pl.ds(0, K)], labels_hbm.at[b])
        pltpu.sync_copy(stage_bx.at[pl.ds(0, 4 * K)], boxes_hbm.at[b])


@jax.jit
def kernel(box_cls, gau_logits):
    m = _masked_scores(box_cls, gau_logits)
    gau_flat = gau_logits.reshape(N * TOT)

    mesh = plsc.VectorSubcoreMesh(core_axis_name="c", subcore_axis_name="s")
    run = pl.kernel(
        _sc_body,
        out_type=(
            jax.ShapeDtypeStruct((N, 4 * K), jnp.float32),
            jax.ShapeDtypeStruct((N, K), jnp.float32),
            jax.ShapeDtypeStruct((N, K), jnp.int32),
        ),
        mesh=mesh,
        compiler_params=pltpu.CompilerParams(
            needs_layout_passes=False, use_tc_tiling_on_sc=False),
        scratch_types=[
            pltpu.VMEM((CH,), jnp.float32),
            pltpu.VMEM((NB,), jnp.int32),
            pltpu.VMEM((NTILES, 256), jnp.int32),
            pltpu.VMEM((256,), jnp.int32),
            pltpu.VMEM((NB,), jnp.int32),
            pltpu.VMEM((16,), jnp.int32),
            pltpu.VMEM((TCAP + 16,), jnp.float32),
            pltpu.VMEM((TCAP + 16,), jnp.int32),
            pltpu.VMEM((GCAP,), jnp.float32),
            pltpu.VMEM((GCAP,), jnp.int32),
            pltpu.VMEM((TCAP,), jnp.int32),
            pltpu.VMEM((TCAP,), jnp.int32),
            pltpu.VMEM((TCAP,), jnp.float32),
            pltpu.VMEM((TCAP,), jnp.float32),
            pltpu.VMEM((TCAP,), jnp.int32),
            pltpu.VMEM((TCAP,), jnp.float32),
            pltpu.VMEM((TCAP,), jnp.float32),
            pltpu.VMEM((TCAP,), jnp.float32),
            pltpu.VMEM((TCAP,), jnp.float32),
            pltpu.VMEM((TCAP,), jnp.int32),
            pltpu.VMEM((1008,), jnp.int32),
            pltpu.VMEM((4032,), jnp.float32),
            pltpu.VMEM_SHARED((NTILES, NB), jnp.int32),
            pltpu.VMEM_SHARED((NB,), jnp.int32),
            pltpu.VMEM_SHARED((GCAP,), jnp.float32),
            pltpu.VMEM_SHARED((GCAP,), jnp.int32),
            pltpu.VMEM_SHARED((16,), jnp.int32),
            pltpu.VMEM_SHARED((1008,), jnp.float32),
            pltpu.VMEM_SHARED((1008,), jnp.int32),
            pltpu.VMEM_SHARED((4032,), jnp.float32),
            pltpu.SemaphoreType.DMA,
        ],
    )
    boxes_flat, scores, labels = run(m, gau_flat)
    return boxes_flat.reshape(N, K, 4), scores, labels


# speedup vs baseline: 18.6016x; 1.6135x over previous
import functools

import jax
import jax.numpy as jnp
from jax import lax
from jax.experimental import pallas as pl
from jax.experimental.pallas import tpu as pltpu
from jax.experimental.pallas import tpu_sc as plsc

N = 2
C = 80
H = 160
W = 160
HW = H * W
TOT = C * HW
PRE = 0.05
K = 1000
STEP = 8
SIGMA2 = 0.125

NB = 4096
NTILES = 16
PER_TILE = TOT // NTILES
CH = 16000
NCH = PER_TILE // CH
TCAP = 128
CAPL = 16384
GCAP = NTILES * TCAP
DUMP = K


def _dense_body(box_ref, gau_ref, out_ref):
    g = gau_ref[...]
    bx = box_ref[...]
    gp = 1.0 / (1.0 + jnp.exp(-g))
    bp = 1.0 / (1.0 + jnp.exp(-bx))
    m1 = jnp.maximum(gp, jnp.maximum(pltpu.roll(gp, 1, 1), pltpu.roll(gp, H - 1, 1)))
    mp = jnp.maximum(m1, jnp.maximum(pltpu.roll(m1, 1, 2), pltpu.roll(m1, W - 1, 2)))
    iy = lax.broadcasted_iota(jnp.int32, g.shape, 1)
    ix = lax.broadcasted_iota(jnp.int32, g.shape, 2)
    interior = (iy > 0) & (iy < H - 1) & (ix > 0) & (ix < W - 1)
    keep = (gp == mp) & (bp > PRE) & interior
    out_ref[...] = jnp.where(keep, jnp.sqrt(gp * bp), jnp.float32(-1.0))


def _masked_scores(box_cls, gau_logits):
    x3 = box_cls.reshape(N * C, H, W)
    g3 = gau_logits.reshape(N * C, H, W)
    m = pl.pallas_call(
        _dense_body,
        grid=(20,),
        in_specs=[pl.BlockSpec((8, H, W), lambda i: (i, 0, 0))] * 2,
        out_specs=pl.BlockSpec((8, H, W), lambda i: (i, 0, 0)),
        out_shape=jax.ShapeDtypeStruct((N * C, H, W), jnp.float32),
    )(x3, g3)
    return m.reshape(N, TOT)


def _sc_body(m_hbm, gau_hbm, boxes_hbm, scores_hbm, labels_hbm,
             chunk_v, hist_v, red_v, redsum_v, hist_full_v, edge_v,
             cl_s, cl_i, cs_v, ci_v, all_s, all_i, ridx_v, gidx_v, gbuf_v,
             sc_v, lb_v, b0_v, b1_v, b2_v, b3_v, bidx_v, pidx_v, cnt_v,
             cntl_v, zbi_v, zbx_v,
             hists_s, hist_red_s, cands_s, candi_s, edge_s, cnts_s,
             stage_sc, stage_lb, stage_bx, sem0, sem1):
    b = lax.axis_index("c")
    sid = lax.axis_index("s")
    base = sid * PER_TILE
    z16i = jnp.zeros((16,), jnp.int32)
    z16f = jnp.zeros((16,), jnp.float32)
    ones16 = jnp.ones((16,), jnp.int32)
    iota16 = lax.broadcasted_iota(jnp.int32, (16,), 0)
    fNB = jnp.float32(NB)

    @pl.when(sid == 0)
    def _p0():
        def zi(i, _):
            zbi_v[pl.ds(i * 16, 16)] = z16i
            return 0
        lax.fori_loop(0, 1008 // 16, zi, 0)

        def zf(i, _):
            zbx_v[pl.ds(i * 16, 16)] = z16f
            return 0
        lax.fori_loop(0, 4032 // 16, zf, 0)
        pltpu.sync_copy(zbx_v.at[pl.ds(0, 1008)], stage_sc)
        pltpu.sync_copy(zbi_v, stage_lb)
        pltpu.sync_copy(zbx_v, stage_bx)

        def zc(i, _):
            all_s[pl.ds(i * 16, 16)] = z16f - 1.0
            all_i[pl.ds(i * 16, 16)] = z16i
            return 0
        lax.fori_loop(0, GCAP // 16, zc, 0)
        pltpu.sync_copy(all_s, cands_s.at[pl.ds(0, GCAP)])
        pltpu.sync_copy(all_i, candi_s.at[pl.ds(0, GCAP)])

    copies = [None, None]
    sems = [sem0, sem1]
    copies[0] = pltpu.async_copy(
        m_hbm.at[b, pl.ds(base, CH)], chunk_v.at[0], sems[0])
    w = jnp.int32(0)
    for k in range(NCH):
        copies[k % 2].wait()
        if k + 1 < NCH:
            copies[(k + 1) % 2] = pltpu.async_copy(
                m_hbm.at[b, pl.ds(base + (k + 1) * CH, CH)],
                chunk_v.at[(k + 1) % 2], sems[(k + 1) % 2])

        def jloop(j, wc, k=k):
            for u in range(8):
                o = (j * 8 + u) * 16
                s = chunk_v[k % 2, pl.ds(o, 16)]
                msk = s >= 0.0
                idxv = base + k * CH + o + iota16
                wclip = jnp.minimum(wc, CAPL)
                plsc.store_compressed(cl_s.at[pl.ds(wclip, 16)], s, mask=msk)
                plsc.store_compressed(cl_i.at[pl.ds(wclip, 16)], idxv, mask=msk)
                wc = wc + plsc.all_reduce_population_count(msk)[0]
            return wc
        w = lax.fori_loop(0, CH // 128, jloop, w)

    def zh(i, _):
        hist_v[pl.ds(i * 16, 16)] = z16i
        return 0
    lax.fori_loop(0, NB // 16, zh, 0)
    nvl = (w + 15) // 16

    def hloop(j, _):
        s = cl_s[pl.ds(j * 16, 16)]
        bins = jnp.clip((s * fNB).astype(jnp.int32), 0, NB - 1)
        inb = (j * 16 + iota16) < w
        plsc.addupdate_scatter(hist_v, [bins], ones16, mask=inb)
        return 0
    lax.fori_loop(0, nvl, hloop, 0)
    pltpu.sync_copy(hist_v, hists_s.at[sid])
    plsc.subcore_barrier()

    for t in range(NTILES):
        pltpu.sync_copy(hists_s.at[t, pl.ds(sid * 256, 256)], red_v.at[t])

    def red_loop(j, _):
        acc = z16i
        for t in range(NTILES):
            acc = acc + red_v[t, pl.ds(j * 16, 16)]
        redsum_v[pl.ds(j * 16, 16)] = acc
        return 0
    lax.fori_loop(0, 256 // 16, red_loop, 0)
    pltpu.sync_copy(redsum_v, hist_red_s.at[pl.ds(sid * 256, 256)])
    plsc.subcore_barrier()

    @pl.when(sid == 0)
    def _p2b():
        pltpu.sync_copy(hist_red_s, hist_full_v)

        def scan_body(j, carry):
            run, found, bst = carry
            vidx = (NB // 16 - 1) - j
            v = hist_full_v[pl.ds(vidx * 16, 16)]
            vd = lax.rev(v, (0,))
            cum = plsc.cumsum(vd) + run
            msk = cum >= K
            anym = jnp.any(msk)
            ffs = jnp.max(plsc.all_reduce_ffs(msk))
            cand = vidx * 16 + 15 - ffs
            take = jnp.logical_and(jnp.logical_not(found), anym)
            return (run + jnp.sum(v),
                    jnp.logical_or(found, anym),
                    jnp.where(take, cand, bst))
        _, found, bst = lax.fori_loop(
            0, NB // 16, scan_body,
            (jnp.int32(0), jnp.bool_(False), jnp.int32(0)))
        bst = jnp.where(found, bst, jnp.int32(0))
        edge_v[...] = jnp.full((16,), 0, jnp.int32) + bst
        pltpu.sync_copy(edge_v, edge_s)
    plsc.subcore_barrier()
    pltpu.sync_copy(edge_s, edge_v)
    bstar = jnp.max(edge_v[...])

    nfilt = jnp.int32(0)

    def floop(j, wc):
        s = cl_s[pl.ds(j * 16, 16)]
        iv = cl_i[pl.ds(j * 16, 16)]
        bins = jnp.clip((s * fNB).astype(jnp.int32), 0, NB - 1)
        inb = (j * 16 + iota16) < w
        msk = jnp.logical_and(inb, bins >= bstar)
        wclip = jnp.minimum(wc, TCAP)
        plsc.store_compressed(cs_v.at[pl.ds(wclip, 16)], s, mask=msk)
        plsc.store_compressed(ci_v.at[pl.ds(wclip, 16)], iv, mask=msk)
        return wc + plsc.all_reduce_population_count(msk)[0]
    nfilt = lax.fori_loop(0, nvl, floop, nfilt)
    nfilt = jnp.minimum(nfilt, jnp.int32(TCAP))

    cnt_v[...] = z16i + nfilt
    pltpu.sync_copy(cnt_v, cnts_s.at[sid])
    plsc.subcore_barrier()
    pltpu.sync_copy(cnts_s, cntl_v)
    off = jnp.int32(0)
    mtot = jnp.int32(0)
    for t in range(NTILES):
        ct = cntl_v[t, pl.ds(0, 16)][0]
        off = off + jnp.where(jnp.int32(t) < sid, ct, 0)
        mtot = mtot + ct
    for u in range(TCAP // 16):
        lanepos = u * 16 + iota16
        dst = jnp.where(lanepos < nfilt, off + lanepos, jnp.int32(GCAP))
        pidx_v[...] = dst
        pltpu.sync_copy(cs_v.at[pl.ds(u * 16, 16)], cands_s.at[pidx_v])
        pltpu.sync_copy(ci_v.at[pl.ds(u * 16, 16)], candi_s.at[pidx_v])
    plsc.subcore_barrier()

    pltpu.sync_copy(cands_s.at[pl.ds(0, GCAP)], all_s)
    pltpu.sync_copy(candi_s.at[pl.ds(0, GCAP)], all_i)
    nv = (jnp.minimum(mtot, jnp.int32(GCAP)) + 15) // 16
    nmy = jnp.maximum((nv - sid + 15) // 16, 0)

    for u in range(TCAP // 16):
        ridx_v[pl.ds(u * 16, 16)] = z16i + DUMP
        cs_v[pl.ds(u * 16, 16)] = z16f - 1.0
        ci_v[pl.ds(u * 16, 16)] = z16i

    def my_loop(m, _):
        j = sid + m * 16
        sjv = all_s[pl.ds(j * 16, 16)]
        ijv = all_i[pl.ds(j * 16, 16)]

        def other_loop(t, rankv):
            va = all_s[pl.ds(t * 16, 16)]
            vi = all_i[pl.ds(t * 16, 16)]
            for lane in range(16):
                s_i = va[lane]
                i_i = vi[lane]
                better = jnp.logical_or(
                    s_i > sjv, jnp.logical_and(s_i == sjv, i_i < ijv))
                rankv = rankv + jnp.where(better, 1, 0)
            return rankv
        rankv = lax.fori_loop(0, nv, other_loop, z16i)
        rankv = jnp.where(
            jnp.logical_and(sjv >= 0.0, rankv < K), rankv, jnp.int32(DUMP))
        ridx_v[pl.ds(m * 16, 16)] = rankv
        cs_v[pl.ds(m * 16, 16)] = sjv
        ci_v[pl.ds(m * 16, 16)] = ijv
        return 0
    lax.fori_loop(0, nmy, my_loop, 0)

    for t in range(TCAP // 16):
        o = t * 16
        sv = cs_v[pl.ds(o, 16)]
        iv = ci_v[pl.ds(o, 16)]
        gidx_v[pl.ds(o, 16)] = jnp.where(sv >= 0.0, iv, 0) + b * TOT
        sc_v[pl.ds(o, 16)] = sv
    pltpu.async_copy(gau_hbm.at[gidx_v], gbuf_v, sem0).wait()

    for t in range(TCAP // 16):
        o = t * 16
        g = gbuf_v[pl.ds(o, 16)]
        iv = ci_v[pl.ds(o, 16)]
        u = jnp.exp(-jnp.abs(g))
        z = u / (2.0 + u)
        z2 = z * z
        l1p = 2.0 * z * (1.0 + z2 * (jnp.float32(1.0 / 3.0) + z2 * (
            jnp.float32(0.2) + z2 * jnp.float32(1.0 / 7.0))))
        li = (jnp.maximum(-g, 0.0) + l1p) * jnp.float32(SIGMA2)
        xc = jnp.maximum(li, jnp.float32(1e-12))
        bi = plsc.bitcast(xc, jnp.int32)
        yi = jnp.int32(0x5F3759DF) - lax.shift_right_arithmetic(bi, 1)
        y = plsc.bitcast(yi, jnp.float32)
        for _ in range(3):
            y = y * (1.5 - 0.5 * xc * y * y)
        halfw = xc * y * jnp.float32(STEP) + 0.5
        ch = iv // HW
        rem = iv % HW
        yy = rem // W
        xx = rem % W
        cx = xx.astype(jnp.float32) * STEP + jnp.float32((STEP - 1) / 2.0)
        cy = yy.astype(jnp.float32) * STEP + jnp.float32((STEP - 1) / 2.0)
        b0_v[pl.ds(o, 16)] = cx - halfw
        b1_v[pl.ds(o, 16)] = cy - halfw
        b2_v[pl.ds(o, 16)] = cx + halfw
        b3_v[pl.ds(o, 16)] = cy + halfw
        lb_v[pl.ds(o, 16)] = ch + 1

    pltpu.sync_copy(sc_v, stage_sc.at[ridx_v])
    pltpu.sync_copy(lb_v, stage_lb.at[ridx_v])
    for kc, bv in ((0, b0_v), (1, b1_v), (2, b2_v), (3, b3_v)):
        for t in range(TCAP // 16):
            o = t * 16
            bidx_v[pl.ds(o, 16)] = ridx_v[pl.ds(o, 16)] * 4 + kc
        pltpu.sync_copy(bv, stage_bx.at[bidx_v])
    plsc.subcore_barrier()

    @pl.when(sid == 0)
    def _p5():
        pltpu.sync_copy(stage_sc.at[pl.ds(0, K)], scores_hbm.at[b])
        pltpu.sync_copy(stage_lb.at[pl.ds(0, K)], labels_hbm.at[b])
        pltpu.sync_copy(stage_bx.at[pl.ds(0, 4 * K)], boxes_hbm.at[b])


@jax.jit
def kernel(box_cls, gau_logits):
    m = _masked_scores(box_cls, gau_logits)
    gau_flat = gau_logits.reshape(N * TOT)

    mesh = plsc.VectorSubcoreMesh(core_axis_name="c", subcore_axis_name="s")
    run = pl.kernel(
        _sc_body,
        out_type=(
            jax.ShapeDtypeStruct((N, 4 * K), jnp.float32),
            jax.ShapeDtypeStruct((N, K), jnp.float32),
            jax.ShapeDtypeStruct((N, K), jnp.int32),
        ),
        mesh=mesh,
        compiler_params=pltpu.CompilerParams(
            needs_layout_passes=False, use_tc_tiling_on_sc=False),
        scratch_types=[
            pltpu.VMEM((2, CH), jnp.float32),
            pltpu.VMEM((NB,), jnp.int32),
            pltpu.VMEM((NTILES, 256), jnp.int32),
            pltpu.VMEM((256,), jnp.int32),
            pltpu.VMEM((NB,), jnp.int32),
            pltpu.VMEM((16,), jnp.int32),
            pltpu.VMEM((CAPL + 16,), jnp.float32),
            pltpu.VMEM((CAPL + 16,), jnp.int32),
            pltpu.VMEM((TCAP + 16,), jnp.float32),
            pltpu.VMEM((TCAP + 16,), jnp.int32),
            pltpu.VMEM((GCAP,), jnp.float32),
            pltpu.VMEM((GCAP,), jnp.int32),
            pltpu.VMEM((TCAP,), jnp.int32),
            pltpu.VMEM((TCAP,), jnp.int32),
            pltpu.VMEM((TCAP,), jnp.float32),
            pltpu.VMEM((TCAP,), jnp.float32),
            pltpu.VMEM((TCAP,), jnp.int32),
            pltpu.VMEM((TCAP,), jnp.float32),
            pltpu.VMEM((TCAP,), jnp.float32),
            pltpu.VMEM((TCAP,), jnp.float32),
            pltpu.VMEM((TCAP,), jnp.float32),
            pltpu.VMEM((TCAP,), jnp.int32),
            pltpu.VMEM((16,), jnp.int32),
            pltpu.VMEM((16,), jnp.int32),
            pltpu.VMEM((16, 16), jnp.int32),
            pltpu.VMEM((1008,), jnp.int32),
            pltpu.VMEM((4032,), jnp.float32),
            pltpu.VMEM_SHARED((NTILES, NB), jnp.int32),
            pltpu.VMEM_SHARED((NB,), jnp.int32),
            pltpu.VMEM_SHARED((GCAP + 16,), jnp.float32),
            pltpu.VMEM_SHARED((GCAP + 16,), jnp.int32),
            pltpu.VMEM_SHARED((16,), jnp.int32),
            pltpu.VMEM_SHARED((16, 16), jnp.int32),
            pltpu.VMEM_SHARED((1008,), jnp.float32),
            pltpu.VMEM_SHARED((1008,), jnp.int32),
            pltpu.VMEM_SHARED((4032,), jnp.float32),
            pltpu.SemaphoreType.DMA,
            pltpu.SemaphoreType.DMA,
        ],
    )
    boxes_flat, scores, labels = run(m, gau_flat)
    return boxes_flat.reshape(N, K, 4), scores, labels
